# baseline (speedup 1.0000x reference)
import functools

import jax
import jax.numpy as jnp
from jax import lax
from jax.experimental import pallas as pl
from jax.experimental.pallas import tpu as pltpu
from jax.experimental.pallas import tpu_sc as plsc

IN_F = 65536
OUT_F = 16384
NCON = 64
NS = 16
HALF = OUT_F // 2
ROWS_PER_W = HALF // NS
GROUPS = ROWS_PER_W // 16


def _make_body(half_base):
    def _body(x_hbm, conn_hbm, op_hbm, out_hbm, x_v, conn_v, op_v, out_v,
              dma_sem):
        base = half_base + lax.axis_index("s") * ROWS_PER_W

        copies = []
        xc = IN_F // 8
        for i in range(8):
            copies.append(pltpu.make_async_copy(
                x_hbm.at[pl.ds(i * xc, xc)], x_v.at[pl.ds(i * xc, xc)],
                dma_sem))
        cc = (ROWS_PER_W * NCON) // 4
        for i in range(4):
            copies.append(pltpu.make_async_copy(
                conn_hbm.at[pl.ds(base * NCON + i * cc, cc)],
                conn_v.at[pl.ds(i * cc, cc)], dma_sem))
        copies.append(pltpu.make_async_copy(
            op_hbm.at[pl.ds(base, ROWS_PER_W)], op_v, dma_sem))
        for c in copies:
            c.start()
        for c in copies:
            c.wait()

        lane = lax.iota(jnp.int32, 16)
        row_off = lane * NCON

        def group(g, carry):
            pos0 = (g * 16) * NCON + row_off

            inf = jnp.full((16,), jnp.inf, jnp.float32)
            amins = [inf] * 4
            amaxs = [-inf] * 4
            for j in range(NCON):
                a = j % 4
                ci = plsc.load_gather(conn_v, [pos0 + j])
                v = plsc.load_gather(x_v, [ci])
                amins[a] = jnp.minimum(amins[a], v)
                amaxs[a] = jnp.maximum(amaxs[a], v)
            mins = jnp.minimum(jnp.minimum(amins[0], amins[1]),
                               jnp.minimum(amins[2], amins[3]))
            maxs = jnp.maximum(jnp.maximum(amaxs[0], amaxs[1]),
                               jnp.maximum(amaxs[2], amaxs[3]))
            opv = op_v[pl.ds(g * 16, 16)]
            out_v[pl.ds(g * 16, 16)] = jnp.where(opv == 0, mins, maxs)
            return carry

        lax.fori_loop(0, GROUPS, group, 0)
        pltpu.sync_copy(out_v, out_hbm.at[pl.ds(base - half_base, ROWS_PER_W)])

    return _body


@jax.jit
def kernel(x, connection_indices, operator_indices):
    conn = connection_indices.reshape(-1).astype(jnp.int32)
    op = operator_indices.astype(jnp.int32)

    halves = []
    for h in range(2):
        mesh = plsc.VectorSubcoreMesh(
            core_axis_name="c", subcore_axis_name="s", num_cores=1)
        call = functools.partial(
            pl.kernel,
            mesh=mesh,
            out_type=jax.ShapeDtypeStruct((HALF,), jnp.float32),
            compiler_params=pltpu.CompilerParams(needs_layout_passes=False),
            scratch_types=[
                pltpu.VMEM((IN_F,), jnp.float32),
                pltpu.VMEM((ROWS_PER_W * NCON,), jnp.int32),
                pltpu.VMEM((ROWS_PER_W,), jnp.int32),
                pltpu.VMEM((ROWS_PER_W,), jnp.float32),
                pltpu.SemaphoreType.DMA,
            ],
        )(_make_body(h * HALF))
        halves.append(call(x, conn, op))
    return jnp.concatenate(halves)


# speedup vs baseline: 1.2255x; 1.2255x over previous
import functools

import jax
import jax.numpy as jnp
from jax import lax
from jax.experimental import pallas as pl
from jax.experimental.pallas import tpu as pltpu
from jax.experimental.pallas import tpu_sc as plsc

IN_F = 65536
OUT_F = 16384
NCON = 64
NS = 16
ROWS_PER_W = OUT_F // NS
NCHUNK = 4
CROWS = ROWS_PER_W // NCHUNK
CWORDS = CROWS * NCON
CGROUPS = CROWS // 16
XSH = IN_F // NS


def _body(x_hbm, conn_hbm, op_hbm, out_hbm, x_sh, x_v, conn_a, conn_b,
          op_v, out_v, sem_x, sem_a, sem_b, sem_op):
    sid = lax.axis_index("s")
    base = sid * ROWS_PER_W

    bufs = (conn_a, conn_b)
    sems = (sem_a, sem_b)

    def chunk_copy(c):
        return pltpu.make_async_copy(
            conn_hbm.at[pl.ds((base + c * CROWS) * NCON, CWORDS)],
            bufs[c % 2], sems[c % 2])

    xstage = pltpu.make_async_copy(
        x_hbm.at[pl.ds(sid * XSH, XSH)], x_sh.at[pl.ds(sid * XSH, XSH)],
        sem_x)
    opcopy = pltpu.make_async_copy(
        op_hbm.at[pl.ds(base, ROWS_PER_W)], op_v, sem_op)
    xstage.start()
    opcopy.start()
    chunk_copy(0).start()
    chunk_copy(1).start()
    xstage.wait()
    plsc.subcore_barrier()

    xfan = pltpu.make_async_copy(x_sh, x_v, sem_x)
    xfan.start()
    xfan.wait()
    opcopy.wait()

    lane = lax.iota(jnp.int32, 16)
    row_off = lane * NCON

    def make_group(buf, crow0):
        def group(g, carry):
            pos0 = (g * 16) * NCON + row_off
            inf = jnp.full((16,), jnp.inf, jnp.float32)
            amins = [inf] * 4
            amaxs = [-inf] * 4
            for j in range(NCON):
                a = j % 4
                ci = plsc.load_gather(buf, [pos0 + j])
                v = plsc.load_gather(x_v, [ci])
                amins[a] = jnp.minimum(amins[a], v)
                amaxs[a] = jnp.maximum(amaxs[a], v)
            mins = jnp.minimum(jnp.minimum(amins[0], amins[1]),
                               jnp.minimum(amins[2], amins[3]))
            maxs = jnp.maximum(jnp.maximum(amaxs[0], amaxs[1]),
                               jnp.maximum(amaxs[2], amaxs[3]))
            opv = op_v[pl.ds(crow0 + g * 16, 16)]
            out_v[pl.ds(crow0 + g * 16, 16)] = jnp.where(opv == 0, mins, maxs)
            return carry
        return group

    for c in range(NCHUNK):
        b = c % 2
        chunk_copy(c).wait()
        lax.fori_loop(0, CGROUPS, make_group(bufs[b], c * CROWS), 0)
        if c + 2 < NCHUNK:
            chunk_copy(c + 2).start()

    pltpu.sync_copy(out_v, out_hbm.at[pl.ds(base, ROWS_PER_W)])


@jax.jit
def kernel(x, connection_indices, operator_indices):
    conn = connection_indices.reshape(-1).astype(jnp.int32)
    op = operator_indices.astype(jnp.int32)

    mesh = plsc.VectorSubcoreMesh(
        core_axis_name="c", subcore_axis_name="s", num_cores=1)
    call = functools.partial(
        pl.kernel,
        mesh=mesh,
        out_type=jax.ShapeDtypeStruct((OUT_F,), jnp.float32),
        compiler_params=pltpu.CompilerParams(needs_layout_passes=False),
        scratch_types=[
            pltpu.VMEM_SHARED((IN_F,), jnp.float32),
            pltpu.VMEM((IN_F,), jnp.float32),
            pltpu.VMEM((CWORDS,), jnp.int32),
            pltpu.VMEM((CWORDS,), jnp.int32),
            pltpu.VMEM((ROWS_PER_W,), jnp.int32),
            pltpu.VMEM((ROWS_PER_W,), jnp.float32),
            pltpu.SemaphoreType.DMA,
            pltpu.SemaphoreType.DMA,
            pltpu.SemaphoreType.DMA,
            pltpu.SemaphoreType.DMA,
        ],
    )(_body)
    return call(x, conn, op)


# speedup vs baseline: 1.4370x; 1.1726x over previous
import functools

import jax
import jax.numpy as jnp
from jax import lax
from jax.experimental import pallas as pl
from jax.experimental.pallas import tpu as pltpu
from jax.experimental.pallas import tpu_sc as plsc

IN_F = 65536
OUT_F = 16384
NCON = 64
NPAIR = NCON // 2
NC = 2
NS = 16
NW = NC * NS
ROWS_PER_W = OUT_F // NW
GROUPS = ROWS_PER_W // 16
XSH = IN_F // NS


def _body(x_hbm, conn_hbm, op_hbm, out_hbm, x_sh, x_v, conn_v, op_v,
          out_v, dma_sem):
    cid = lax.axis_index("c")
    sid = lax.axis_index("s")
    base = (cid * NS + sid) * ROWS_PER_W

    stage = [
        pltpu.make_async_copy(
            x_hbm.at[pl.ds(sid * XSH, XSH)], x_sh.at[pl.ds(sid * XSH, XSH)],
            dma_sem),
        pltpu.make_async_copy(
            conn_hbm.at[pl.ds(base * NPAIR, ROWS_PER_W * NPAIR)], conn_v,
            dma_sem),
        pltpu.make_async_copy(
            op_hbm.at[pl.ds(base, ROWS_PER_W)], op_v, dma_sem),
    ]
    for c in stage:
        c.start()
    stage[0].wait()
    plsc.subcore_barrier()

    xcopy = pltpu.make_async_copy(x_sh, x_v, dma_sem)
    xcopy.start()
    xcopy.wait()
    stage[1].wait()
    stage[2].wait()

    lane = lax.iota(jnp.int32, 16)
    row_off = lane * NPAIR

    def group(g, carry):
        pos0 = (g * 16) * NPAIR + row_off

        inf = jnp.full((16,), jnp.inf, jnp.float32)
        amins = [inf] * 4
        amaxs = [-inf] * 4
        for j in range(NPAIR):
            w = plsc.load_gather(conn_v, [pos0 + j])
            lo = jnp.bitwise_and(w, 0xFFFF)
            hi = lax.shift_right_logical(w, 16)
            v0 = plsc.load_gather(x_v, [lo])
            v1 = plsc.load_gather(x_v, [hi])
            a = (2 * j) % 4
            b = (2 * j + 1) % 4
            amins[a] = jnp.minimum(amins[a], v0)
            amaxs[a] = jnp.maximum(amaxs[a], v0)
            amins[b] = jnp.minimum(amins[b], v1)
            amaxs[b] = jnp.maximum(amaxs[b], v1)
        mins = jnp.minimum(jnp.minimum(amins[0], amins[1]),
                           jnp.minimum(amins[2], amins[3]))
        maxs = jnp.maximum(jnp.maximum(amaxs[0], amaxs[1]),
                           jnp.maximum(amaxs[2], amaxs[3]))
        opv = op_v[pl.ds(g * 16, 16)]
        out_v[pl.ds(g * 16, 16)] = jnp.where(opv == 0, mins, maxs)
        return carry

    lax.fori_loop(0, GROUPS, group, 0)
    pltpu.sync_copy(out_v, out_hbm.at[pl.ds(base, ROWS_PER_W)])


@jax.jit
def kernel(x, connection_indices, operator_indices):
    conn_u16 = connection_indices.astype(jnp.uint16)
    conn = lax.bitcast_convert_type(
        conn_u16.reshape(OUT_F, NPAIR, 2), jnp.int32).reshape(-1)
    op = operator_indices.astype(jnp.int32)

    mesh = plsc.VectorSubcoreMesh(core_axis_name="c", subcore_axis_name="s")
    call = functools.partial(
        pl.kernel,
        mesh=mesh,
        out_type=jax.ShapeDtypeStruct((OUT_F,), jnp.float32),
        compiler_params=pltpu.CompilerParams(needs_layout_passes=False),
        scratch_types=[
            pltpu.VMEM_SHARED((IN_F,), jnp.float32),
            pltpu.VMEM((IN_F,), jnp.float32),
            pltpu.VMEM((ROWS_PER_W * NPAIR,), jnp.int32),
            pltpu.VMEM((ROWS_PER_W,), jnp.int32),
            pltpu.VMEM((ROWS_PER_W,), jnp.float32),
            pltpu.SemaphoreType.DMA,
        ],
    )(_body)
    return call(x, conn, op)


# speedup vs baseline: 1.5301x; 1.0647x over previous
import functools

import jax
import jax.numpy as jnp
from jax import lax
from jax.experimental import pallas as pl
from jax.experimental.pallas import tpu as pltpu
from jax.experimental.pallas import tpu_sc as plsc

IN_F = 65536
OUT_F = 16384
NCON = 64
NC = 2
NS = 16
NW = NC * NS
ROWS_PER_W = OUT_F // NW
ROWS_PER_C = OUT_F // NC
GROUPS = ROWS_PER_W // 16
XSH = IN_F // NS


def _body(x_hbm, conn_hbm, op_hbm, out_hbm, x_sh, x_v, conn_v, op_v,
          out_v, dma_sem):
    cid = lax.axis_index("c")
    sid = lax.axis_index("s")
    base = (cid * NS + sid) * ROWS_PER_W
    cslice = sid * ROWS_PER_W * NCON

    del cslice
    stage = [
        pltpu.make_async_copy(
            x_hbm.at[pl.ds(sid * XSH, XSH)], x_sh.at[pl.ds(sid * XSH, XSH)],
            dma_sem),
        pltpu.make_async_copy(
            conn_hbm.at[pl.ds(base * NCON, ROWS_PER_W * NCON)], conn_v,
            dma_sem),
        pltpu.make_async_copy(
            op_hbm.at[pl.ds(base, ROWS_PER_W)], op_v, dma_sem),
    ]
    for c in stage:
        c.start()
    for c in stage:
        c.wait()
    plsc.subcore_barrier()

    xcopy = pltpu.make_async_copy(x_sh, x_v, dma_sem)
    xcopy.start()
    xcopy.wait()

    lane = lax.iota(jnp.int32, 16)
    row_off = lane * NCON

    def group(g, carry):
        pos0 = (g * 16) * NCON + row_off

        inf = jnp.full((16,), jnp.inf, jnp.float32)
        amins = [inf] * 4
        amaxs = [-inf] * 4
        for j in range(NCON):
            a = j % 4
            ci = plsc.load_gather(conn_v, [pos0 + j])
            v = plsc.load_gather(x_v, [ci])
            amins[a] = jnp.minimum(amins[a], v)
            amaxs[a] = jnp.maximum(amaxs[a], v)
        mins = jnp.minimum(jnp.minimum(amins[0], amins[1]),
                           jnp.minimum(amins[2], amins[3]))
        maxs = jnp.maximum(jnp.maximum(amaxs[0], amaxs[1]),
                           jnp.maximum(amaxs[2], amaxs[3]))
        opv = op_v[pl.ds(g * 16, 16)]
        out_v[pl.ds(g * 16, 16)] = jnp.where(opv == 0, mins, maxs)
        return carry

    lax.fori_loop(0, GROUPS, group, 0)
    pltpu.sync_copy(out_v, out_hbm.at[pl.ds(base, ROWS_PER_W)])


@jax.jit
def kernel(x, connection_indices, operator_indices):
    conn = connection_indices.reshape(-1).astype(jnp.int32)
    op = operator_indices.astype(jnp.int32)

    mesh = plsc.VectorSubcoreMesh(core_axis_name="c", subcore_axis_name="s")
    call = functools.partial(
        pl.kernel,
        mesh=mesh,
        out_type=jax.ShapeDtypeStruct((OUT_F,), jnp.float32),
        compiler_params=pltpu.CompilerParams(needs_layout_passes=False),
        scratch_types=[
            pltpu.VMEM_SHARED((IN_F,), jnp.float32),
            pltpu.VMEM((IN_F,), jnp.float32),
            pltpu.VMEM((ROWS_PER_W * NCON,), jnp.int32),
            pltpu.VMEM((ROWS_PER_W,), jnp.int32),
            pltpu.VMEM((ROWS_PER_W,), jnp.float32),
            pltpu.SemaphoreType.DMA,
        ],
    )(_body)
    return call(x, conn, op)


# speedup vs baseline: 1.5599x; 1.0195x over previous
import functools

import jax
import jax.numpy as jnp
from jax import lax
from jax.experimental import pallas as pl
from jax.experimental.pallas import tpu as pltpu
from jax.experimental.pallas import tpu_sc as plsc

IN_F = 65536
OUT_F = 16384
NCON = 64
NC = 2
NS = 16
NW = NC * NS
ROWS_PER_W = OUT_F // NW
ROWS_PER_C = OUT_F // NC
GROUPS = ROWS_PER_W // 16
XSH = IN_F // NS


def _body(x_hbm, conn_hbm, op_hbm, out_hbm, x_sh, x_v, conn_v, op_v,
          out_v, sem_x, sem_conn, sem_op):
    cid = lax.axis_index("c")
    sid = lax.axis_index("s")
    base = (cid * NS + sid) * ROWS_PER_W

    xstage = pltpu.make_async_copy(
        x_hbm.at[pl.ds(sid * XSH, XSH)], x_sh.at[pl.ds(sid * XSH, XSH)],
        sem_x)
    conncopy = pltpu.make_async_copy(
        conn_hbm.at[pl.ds(base * NCON, ROWS_PER_W * NCON)], conn_v, sem_conn)
    opcopy = pltpu.make_async_copy(
        op_hbm.at[pl.ds(base, ROWS_PER_W)], op_v, sem_op)
    xstage.start()
    conncopy.start()
    opcopy.start()
    xstage.wait()
    plsc.subcore_barrier()

    xcopy = pltpu.make_async_copy(x_sh, x_v, sem_x)
    xcopy.start()
    xcopy.wait()
    conncopy.wait()
    opcopy.wait()

    lane = lax.iota(jnp.int32, 16)
    row_off = lane * NCON

    def group(g, carry):
        pos0 = (g * 16) * NCON + row_off

        inf = jnp.full((16,), jnp.inf, jnp.float32)
        amins = [inf] * 4
        amaxs = [-inf] * 4
        for j in range(NCON):
            a = j % 4
            ci = plsc.load_gather(conn_v, [pos0 + j])
            v = plsc.load_gather(x_v, [ci])
            amins[a] = jnp.minimum(amins[a], v)
            amaxs[a] = jnp.maximum(amaxs[a], v)
        mins = jnp.minimum(jnp.minimum(amins[0], amins[1]),
                           jnp.minimum(amins[2], amins[3]))
        maxs = jnp.maximum(jnp.maximum(amaxs[0], amaxs[1]),
                           jnp.maximum(amaxs[2], amaxs[3]))
        opv = op_v[pl.ds(g * 16, 16)]
        out_v[pl.ds(g * 16, 16)] = jnp.where(opv == 0, mins, maxs)
        return carry

    lax.fori_loop(0, GROUPS, group, 0)
    pltpu.sync_copy(out_v, out_hbm.at[pl.ds(base, ROWS_PER_W)])


@jax.jit
def kernel(x, connection_indices, operator_indices):
    conn = connection_indices.reshape(-1).astype(jnp.int32)
    op = operator_indices.astype(jnp.int32)

    mesh = plsc.VectorSubcoreMesh(core_axis_name="c", subcore_axis_name="s")
    call = functools.partial(
        pl.kernel,
        mesh=mesh,
        out_type=jax.ShapeDtypeStruct((OUT_F,), jnp.float32),
        compiler_params=pltpu.CompilerParams(needs_layout_passes=False),
        scratch_types=[
            pltpu.VMEM_SHARED((IN_F,), jnp.float32),
            pltpu.VMEM((IN_F,), jnp.float32),
            pltpu.VMEM((ROWS_PER_W * NCON,), jnp.int32),
            pltpu.VMEM((ROWS_PER_W,), jnp.int32),
            pltpu.VMEM((ROWS_PER_W,), jnp.float32),
            pltpu.SemaphoreType.DMA,
            pltpu.SemaphoreType.DMA,
            pltpu.SemaphoreType.DMA,
        ],
    )(_body)
    return call(x, conn, op)
